# baseline (speedup 1.0000x reference)
import functools

import jax
import jax.numpy as jnp
from jax import lax
from jax.experimental import pallas as pl
from jax.experimental.pallas import tpu as pltpu
from jax.experimental.pallas import tpu_sc as plsc

_VOCAB = 1000
_VPAD = 1024
_NSLICE = _VPAD // 128
_EMB = 128
_ROWS = 256
_NBUF = 8
_NEG = float("-inf")


def _argmax_rows(xb):
    lane = lax.broadcasted_iota(jnp.int32, (_ROWS, 128), 1)
    acc_v = xb[:, 0:128]
    acc_i = lane
    for k in range(1, _NSLICE):
        lo = k * 128
        v = xb[:, lo:min(lo + 128, _VOCAB)]
        if v.shape[1] < 128:
            v = jnp.concatenate(
                [v, jnp.full((_ROWS, 128 - v.shape[1]), _NEG, jnp.float32)],
                axis=1)
        take = v > acc_v
        acc_v = jnp.where(take, v, acc_v)
        acc_i = jnp.where(take, lane + lo, acc_i)
    m = jnp.max(acc_v, axis=1, keepdims=True)
    cand = jnp.where(acc_v == m, acc_i, _VPAD)
    return jnp.min(cand, axis=1)


def _argmax_body(x_hbm, tok_ref, *scratch):
    bufs = scratch[:_NBUF]
    sems = scratch[_NBUF:]
    nchunks = x_hbm.shape[0] // _ROWS

    def dma(g, slot):
        return pltpu.make_async_copy(
            x_hbm.at[pl.ds(g * _ROWS, _ROWS)], bufs[slot], sems[slot])

    for s in range(_NBUF - 1):
        dma(s, s).start()

    def outer(g0, carry):
        for b in range(_NBUF):
            g = g0 * _NBUF + b
            nxt = g + _NBUF - 1

            @pl.when(nxt < nchunks)
            def _():
                dma(nxt, (b + _NBUF - 1) % _NBUF).start()

            dma(g, b).wait()
            tok_ref[pl.ds(g * _ROWS, _ROWS)] = _argmax_rows(bufs[b][...])
        return carry

    lax.fori_loop(0, nchunks // _NBUF, outer, 0)


def _argmax_tokens(x):
    b, n, v = x.shape
    xf = x.reshape(b * n, v)
    return pl.pallas_call(
        _argmax_body,
        in_specs=[pl.BlockSpec(memory_space=pl.ANY)],
        out_specs=pl.BlockSpec(memory_space=pltpu.VMEM),
        out_shape=jax.ShapeDtypeStruct((b * n,), jnp.int32),
        scratch_shapes=(
            [pltpu.VMEM((_ROWS, _VOCAB), jnp.float32) for _ in range(_NBUF)]
            + [pltpu.SemaphoreType.DMA for _ in range(_NBUF)]
        ),
    )(xf).reshape(b, n)


@functools.cache
def _make_gather(b, n):
    info = plsc.get_sparse_core_info()
    nw = info.num_cores * info.num_subcores
    b_per_w = b // nw
    mesh = plsc.VectorSubcoreMesh(core_axis_name="c", subcore_axis_name="s")

    @functools.partial(
        pl.kernel,
        mesh=mesh,
        out_type=jax.ShapeDtypeStruct((b, n, _EMB), jnp.float32),
        scratch_types=[
            pltpu.VMEM((b_per_w, n), jnp.int32),
            pltpu.VMEM((n, _EMB), jnp.float32),
            pltpu.SemaphoreType.DMA,
        ],
    )
    def gk(tok_hbm, table_hbm, out_hbm, idx_v, rows_v, sem):
        wid = lax.axis_index("s") * info.num_cores + lax.axis_index("c")
        base = wid * b_per_w
        pltpu.sync_copy(tok_hbm.at[pl.ds(base, b_per_w)], idx_v)

        def body(j, carry):
            pltpu.async_copy(table_hbm.at[idx_v.at[j]], rows_v, sem).wait()
            pltpu.sync_copy(rows_v, out_hbm.at[base + j])
            return carry

        lax.fori_loop(0, b_per_w, body, 0)

    return gk


def kernel(x, dictionary):
    b, n, v = x.shape
    tokens = _argmax_tokens(x)
    return _make_gather(b, n)(tokens, dictionary)


# speedup vs baseline: 1.4177x; 1.4177x over previous
import functools

import jax
import jax.numpy as jnp
from jax import lax
from jax.experimental import pallas as pl
from jax.experimental.pallas import tpu as pltpu
from jax.experimental.pallas import tpu_sc as plsc

_VOCAB = 1000
_EMB = 128
_CB = 32
_NBUF = 4


def _argmax_chunk(xb):
    m = jnp.max(xb, axis=2, keepdims=True)
    iota = lax.broadcasted_iota(jnp.int32, xb.shape, 2)
    cand = jnp.where(xb == m, iota, _VOCAB)
    return jnp.min(cand, axis=2)


def _argmax_body(x_hbm, tok_ref, *scratch):
    bufs = scratch[:_NBUF]
    sems = scratch[_NBUF:]
    nchunks = x_hbm.shape[0] // _CB

    def dma(g, slot):
        return pltpu.make_async_copy(
            x_hbm.at[pl.ds(g * _CB, _CB)], bufs[slot], sems[slot])

    for s in range(_NBUF - 1):
        dma(s, s).start()

    def outer(g0, carry):
        for b in range(_NBUF):
            g = g0 * _NBUF + b
            nxt = g + _NBUF - 1

            @pl.when(nxt < nchunks)
            def _():
                dma(nxt, (b + _NBUF - 1) % _NBUF).start()

            dma(g, b).wait()
            tok_ref[pl.ds(g * _CB, _CB), :] = _argmax_chunk(bufs[b][...])
        return carry

    lax.fori_loop(0, nchunks // _NBUF, outer, 0)


def _argmax_tokens(x):
    b, n, v = x.shape
    return pl.pallas_call(
        _argmax_body,
        in_specs=[pl.BlockSpec(memory_space=pl.ANY)],
        out_specs=pl.BlockSpec(memory_space=pltpu.VMEM),
        out_shape=jax.ShapeDtypeStruct((b, n), jnp.int32),
        scratch_shapes=(
            [pltpu.VMEM((_CB, n, v), jnp.float32) for _ in range(_NBUF)]
            + [pltpu.SemaphoreType.DMA for _ in range(_NBUF)]
        ),
    )(x)


@functools.cache
def _make_gather(b, n):
    info = plsc.get_sparse_core_info()
    nw = info.num_cores * info.num_subcores
    b_per_w = b // nw
    mesh = plsc.VectorSubcoreMesh(core_axis_name="c", subcore_axis_name="s")

    @functools.partial(
        pl.kernel,
        mesh=mesh,
        out_type=jax.ShapeDtypeStruct((b, n, _EMB), jnp.float32),
        scratch_types=[
            pltpu.VMEM((b_per_w, n), jnp.int32),
            pltpu.VMEM((n, _EMB), jnp.float32),
            pltpu.VMEM((n, _EMB), jnp.float32),
            pltpu.SemaphoreType.DMA,
            pltpu.SemaphoreType.DMA,
        ],
    )
    def gk(tok_hbm, table_hbm, out_hbm, idx_v, rows0, rows1, sem0, sem1):
        wid = lax.axis_index("s") * info.num_cores + lax.axis_index("c")
        base = wid * b_per_w
        pltpu.sync_copy(tok_hbm.at[pl.ds(base, b_per_w)], idx_v)

        rows = (rows0, rows1)
        sems = (sem0, sem1)

        def gather(j):
            return pltpu.async_copy(
                table_hbm.at[idx_v.at[j]], rows[j % 2], sems[j % 2])

        pend = gather(0)
        for j in range(b_per_w):
            nxt_pend = gather(j + 1) if j + 1 < b_per_w else None
            pend.wait()
            pltpu.sync_copy(rows[j % 2], out_hbm.at[base + j])
            pend = nxt_pend

    return gk


def kernel(x, dictionary):
    b, n, v = x.shape
    tokens = _argmax_tokens(x)
    return _make_gather(b, n)(tokens, dictionary)
